# baseline (speedup 1.0000x reference)
import jax
import jax.numpy as jnp
from jax import lax
from jax.experimental import pallas as pl
from jax.experimental.pallas import tpu as pltpu

_B, _QL, _KVL, _H, _DH = 8, 32, 2048, 16, 128
_ROW = _H * _DH
_BLK = 256
_NJ = _KVL // _BLK


def _cache_kernel(ci_ref, ck_ref, k_ref, cv_ref, v_ref, am_ref,
                  nk_ref, nv_ref, m_ref):
    j = pl.program_id(1)
    base = j * _BLK
    ci = ci_ref[0]
    ci_u = jnp.clip(ci, 0, _KVL - _QL)
    rows = base + lax.broadcasted_iota(jnp.int32, (_BLK, 1), 0)
    sel = (rows >= ci_u) & (rows < ci_u + _QL)
    oh = (rows - ci_u == lax.broadcasted_iota(jnp.int32, (_BLK, _QL), 1)
          ).astype(jnp.float32)
    upd_k = lax.dot(oh, k_ref[0], preferred_element_type=jnp.float32,
                    precision=lax.Precision.HIGHEST)
    upd_v = lax.dot(oh, v_ref[0], preferred_element_type=jnp.float32,
                    precision=lax.Precision.HIGHEST)
    nk_ref[0] = jnp.where(sel, upd_k, ck_ref[0])
    nv_ref[0] = jnp.where(sel, upd_v, cv_ref[0])
    cols = base + lax.broadcasted_iota(jnp.int32, (_QL, _BLK), 1)
    m_ref[0, 0] = am_ref[0, 0] & (cols < ci + _QL)


def kernel(key, value, query_states, attention_mask, cached_key,
           cached_value, cache_index):
    ci = jnp.asarray(cache_index, jnp.int32).reshape((1,))
    ck = cached_key.reshape(_B, _KVL, _ROW)
    cv = cached_value.reshape(_B, _KVL, _ROW)
    k2 = key.reshape(_B, _QL, _ROW)
    v2 = value.reshape(_B, _QL, _ROW)
    grid_spec = pltpu.PrefetchScalarGridSpec(
        num_scalar_prefetch=1,
        grid=(_B, _NJ),
        in_specs=[
            pl.BlockSpec((1, _BLK, _ROW), lambda b, j, ci: (b, j, 0)),
            pl.BlockSpec((1, _QL, _ROW), lambda b, j, ci: (b, 0, 0)),
            pl.BlockSpec((1, _BLK, _ROW), lambda b, j, ci: (b, j, 0)),
            pl.BlockSpec((1, _QL, _ROW), lambda b, j, ci: (b, 0, 0)),
            pl.BlockSpec((1, 1, _QL, _BLK), lambda b, j, ci: (b, 0, 0, j)),
        ],
        out_specs=[
            pl.BlockSpec((1, _BLK, _ROW), lambda b, j, ci: (b, j, 0)),
            pl.BlockSpec((1, _BLK, _ROW), lambda b, j, ci: (b, j, 0)),
            pl.BlockSpec((1, 1, _QL, _BLK), lambda b, j, ci: (b, 0, 0, j)),
        ],
    )
    nk, nv, m = pl.pallas_call(
        _cache_kernel,
        grid_spec=grid_spec,
        out_shape=[
            jax.ShapeDtypeStruct((_B, _KVL, _ROW), jnp.float32),
            jax.ShapeDtypeStruct((_B, _KVL, _ROW), jnp.float32),
            jax.ShapeDtypeStruct((_B, 1, _QL, _KVL), jnp.bool_),
        ],
        compiler_params=pltpu.CompilerParams(
            dimension_semantics=("parallel", "arbitrary")),
    )(ci, ck, k2, cv, v2, attention_mask)
    return (nk.reshape(_B, _KVL, _H, _DH),
            nv.reshape(_B, _KVL, _H, _DH),
            m)


# speedup vs baseline: 1.0698x; 1.0698x over previous
import jax
import jax.numpy as jnp
from jax import lax
from jax.experimental import pallas as pl
from jax.experimental.pallas import tpu as pltpu

_B, _QL, _KVL, _H, _DH = 8, 32, 2048, 16, 128
_ROW = _H * _DH
_BLK = 256
_NJ = _KVL // _BLK


def _cache_kernel(ci_ref, ck_ref, k_ref, cv_ref, v_ref, am_ref,
                  nk_ref, nv_ref, m_ref):
    j = pl.program_id(1)
    base = j * _BLK
    ci = ci_ref[0]
    ci_u = jnp.clip(ci, 0, _KVL - _QL)
    overlap = (ci_u + _QL > base) & (ci_u < base + _BLK)

    @pl.when(jnp.logical_not(overlap))
    def _copy():
        nk_ref[...] = ck_ref[...]
        nv_ref[...] = cv_ref[...]

    @pl.when(overlap)
    def _update():
        rows = base + lax.broadcasted_iota(jnp.int32, (_BLK, 1), 0)
        sel = (rows >= ci_u) & (rows < ci_u + _QL)
        oh = (rows - ci_u == lax.broadcasted_iota(jnp.int32, (_BLK, _QL), 1)
              ).astype(jnp.float32)
        upd_k = lax.dot(oh, k_ref[0], preferred_element_type=jnp.float32,
                        precision=lax.Precision.HIGHEST)
        upd_v = lax.dot(oh, v_ref[0], preferred_element_type=jnp.float32,
                        precision=lax.Precision.HIGHEST)
        nk_ref[0] = jnp.where(sel, upd_k, ck_ref[0])
        nv_ref[0] = jnp.where(sel, upd_v, cv_ref[0])
    cols = base + lax.broadcasted_iota(jnp.int32, (_QL, _BLK), 1)
    m_ref[0, 0] = am_ref[0, 0] & (cols < ci + _QL)


def kernel(key, value, query_states, attention_mask, cached_key,
           cached_value, cache_index):
    ci = jnp.asarray(cache_index, jnp.int32).reshape((1,))
    ck = cached_key.reshape(_B, _KVL, _ROW)
    cv = cached_value.reshape(_B, _KVL, _ROW)
    k2 = key.reshape(_B, _QL, _ROW)
    v2 = value.reshape(_B, _QL, _ROW)
    grid_spec = pltpu.PrefetchScalarGridSpec(
        num_scalar_prefetch=1,
        grid=(_B, _NJ),
        in_specs=[
            pl.BlockSpec((1, _BLK, _ROW), lambda b, j, ci: (b, j, 0)),
            pl.BlockSpec((1, _QL, _ROW), lambda b, j, ci: (b, 0, 0)),
            pl.BlockSpec((1, _BLK, _ROW), lambda b, j, ci: (b, j, 0)),
            pl.BlockSpec((1, _QL, _ROW), lambda b, j, ci: (b, 0, 0)),
            pl.BlockSpec((1, 1, _QL, _BLK), lambda b, j, ci: (b, 0, 0, j)),
        ],
        out_specs=[
            pl.BlockSpec((1, _BLK, _ROW), lambda b, j, ci: (b, j, 0)),
            pl.BlockSpec((1, _BLK, _ROW), lambda b, j, ci: (b, j, 0)),
            pl.BlockSpec((1, 1, _QL, _BLK), lambda b, j, ci: (b, 0, 0, j)),
        ],
    )
    nk, nv, m = pl.pallas_call(
        _cache_kernel,
        grid_spec=grid_spec,
        out_shape=[
            jax.ShapeDtypeStruct((_B, _KVL, _ROW), jnp.float32),
            jax.ShapeDtypeStruct((_B, _KVL, _ROW), jnp.float32),
            jax.ShapeDtypeStruct((_B, 1, _QL, _KVL), jnp.bool_),
        ],
        compiler_params=pltpu.CompilerParams(
            dimension_semantics=("parallel", "arbitrary")),
    )(ci, ck, k2, cv, v2, attention_mask)
    return (nk.reshape(_B, _KVL, _H, _DH),
            nv.reshape(_B, _KVL, _H, _DH),
            m)
